# baseline (speedup 1.0000x reference)
import functools

import jax
import jax.numpy as jnp
from jax import lax
from jax.experimental import pallas as pl
from jax.experimental.pallas import tpu as pltpu
from jax.experimental.pallas import tpu_sc as plsc

N = 10000
E = 320000
D = 128
NC = 2
NS = 16
NW = NC * NS
EPT = E // NW
CH = 125
NCHUNK = EPT // CH
G = 16
NG = NCHUNK // G
NP = 10240
RPT = NP // NS

_MESH = plsc.VectorSubcoreMesh(core_axis_name="c", subcore_axis_name="s")


def _sc_degree(dst_r, ones_ch, zeros_n):
    @functools.partial(
        pl.kernel,
        mesh=_MESH,
        out_type=jax.ShapeDtypeStruct((NC, 1, N), jnp.float32),
        scratch_types=[
            pltpu.VMEM((NCHUNK, CH), jnp.int32),
            pltpu.VMEM((CH,), jnp.float32),
            pltpu.VMEM_SHARED((N,), jnp.float32),
        ],
    )
    def k(dst_hbm, ones_hbm, zeros_hbm, out_hbm, idx_v, ones_v, acc_s):
        cid = lax.axis_index("c")
        sid = lax.axis_index("s")
        wid = sid * NC + cid

        @pl.when(sid == 0)
        def _():
            pltpu.sync_copy(zeros_hbm, acc_s)

        pltpu.sync_copy(ones_hbm, ones_v)
        pltpu.sync_copy(dst_hbm.at[wid], idx_v)
        plsc.subcore_barrier()

        @pl.loop(0, NCHUNK)
        def _(j):
            pltpu.sync_copy(ones_v, acc_s.at[idx_v.at[j]], add=True)

        plsc.subcore_barrier()

        @pl.when(sid == 0)
        def _():
            pltpu.sync_copy(acc_s, out_hbm.at[cid, 0])

    return k(dst_r, ones_ch, zeros_n)


def _sc_scatter(hp, src_r, dst_r, zeros_rpt):
    @functools.partial(
        pl.kernel,
        mesh=_MESH,
        out_type=jax.ShapeDtypeStruct((NC, NP, D), jnp.float32),
        scratch_types=[
            pltpu.VMEM((G, CH), jnp.int32),
            pltpu.VMEM((G, CH), jnp.int32),
            pltpu.VMEM((G, CH), jnp.int32),
            pltpu.VMEM((G, CH), jnp.int32),
            pltpu.VMEM((CH, D), jnp.float32),
            pltpu.VMEM((CH, D), jnp.float32),
            pltpu.VMEM_SHARED((NP, D), jnp.float32),
            pltpu.SemaphoreType.DMA,
            pltpu.SemaphoreType.DMA,
            pltpu.SemaphoreType.DMA,
        ],
    )
    def k(hp_hbm, src_hbm, dst_hbm, z_hbm, out_hbm,
          sb0, sb1, db0, db1, rows0, rows1, acc, gsem, isem, ssem):
        cid = lax.axis_index("c")
        sid = lax.axis_index("s")
        wid = sid * NC + cid
        base = sid * RPT
        sbufs, dbufs = (sb0, sb1), (db0, db1)

        def idx_start(g, b):
            pltpu.async_copy(src_hbm.at[wid, pl.ds(g * G, G)], sbufs[b], isem)
            pltpu.async_copy(dst_hbm.at[wid, pl.ds(g * G, G)], dbufs[b], isem)

        def idx_wait(g, b):
            pltpu.make_async_copy(
                src_hbm.at[wid, pl.ds(g * G, G)], sbufs[b], isem).wait()
            pltpu.make_async_copy(
                dst_hbm.at[wid, pl.ds(g * G, G)], dbufs[b], isem).wait()

        idx_start(0, 0)
        pltpu.sync_copy(z_hbm, acc.at[pl.ds(base, RPT), :])
        plsc.subcore_barrier()

        for g in range(NG):
            b = g % 2
            idx_wait(g, b)
            if g + 1 < NG:
                idx_start(g + 1, (g + 1) % 2)
            sb, db = sbufs[b], dbufs[b]
            hp_c = hp_hbm.at[cid]
            pltpu.async_copy(hp_c.at[sb.at[0]], rows0, gsem)
            if g > 0:
                pdb = dbufs[(g - 1) % 2]
                pltpu.make_async_copy(
                    rows1, acc.at[pdb.at[G - 1]], ssem).wait()

            @pl.loop(0, G, step=2)
            def _(j):
                pltpu.make_async_copy(hp_c.at[sb.at[j]], rows0, gsem).wait()
                pltpu.async_copy(rows0, acc.at[db.at[j]], ssem, add=True)

                @pl.when(j > 0)
                def _():
                    pltpu.make_async_copy(
                        rows1, acc.at[db.at[j - 1]], ssem).wait()

                pltpu.async_copy(hp_c.at[sb.at[j + 1]], rows1, gsem)
                pltpu.make_async_copy(
                    hp_c.at[sb.at[j + 1]], rows1, gsem).wait()
                pltpu.async_copy(rows1, acc.at[db.at[j + 1]], ssem, add=True)
                pltpu.make_async_copy(rows0, acc.at[db.at[j]], ssem).wait()

                @pl.when(j + 2 < G)
                def _():
                    pltpu.async_copy(hp_c.at[sb.at[j + 2]], rows0, gsem)

        pltpu.make_async_copy(
            rows1, acc.at[dbufs[(NG - 1) % 2].at[G - 1]], ssem).wait()
        plsc.subcore_barrier()
        pltpu.sync_copy(acc.at[pl.ds(base, RPT), :],
                        out_hbm.at[cid, pl.ds(base, RPT), :])

    return k(hp, src_r, dst_r, zeros_rpt)


_BLK = 2000


def _tc_matmul(x, W):
    def body(x_ref, w_ref, o_ref):
        o_ref[...] = jnp.dot(x_ref[...], w_ref[...],
                             preferred_element_type=jnp.float32)

    return pl.pallas_call(
        body,
        grid=(N // _BLK,),
        in_specs=[
            pl.BlockSpec((_BLK, D), lambda i: (i, 0)),
            pl.BlockSpec((D, D), lambda i: (0, 0)),
        ],
        out_specs=pl.BlockSpec((_BLK, D), lambda i: (i, 0)),
        out_shape=jax.ShapeDtypeStruct((N, D), jnp.float32),
    )(x, W)


def _tc_scale(h, degp2):
    def body(h_ref, d_ref, o_ref):
        deg = 1.0 + d_ref[..., 0] + d_ref[..., 1]
        o_ref[0] = h_ref[...] * lax.rsqrt(deg)[:, None]

    return pl.pallas_call(
        body,
        grid=(NC, N // _BLK),
        in_specs=[
            pl.BlockSpec((_BLK, D), lambda c, i: (i, 0)),
            pl.BlockSpec((_BLK, 2), lambda c, i: (i, 0)),
        ],
        out_specs=pl.BlockSpec((1, _BLK, D), lambda c, i: (c, i, 0)),
        out_shape=jax.ShapeDtypeStruct((NC, N, D), jnp.float32),
    )(h, degp2)


def _tc_finalize(sp, hp, degp2, b2):
    def body(s0_ref, s1_ref, hp_ref, d_ref, b_ref, o_ref):
        deg = 1.0 + d_ref[..., 0] + d_ref[..., 1]
        dinv = lax.rsqrt(deg)[:, None]
        acc = (s0_ref[0] + s1_ref[0] + hp_ref[0]) * dinv + b_ref[...]
        o_ref[...] = jnp.tanh(acc)

    return pl.pallas_call(
        body,
        grid=(N // _BLK,),
        in_specs=[
            pl.BlockSpec((1, _BLK, D), lambda i: (0, i, 0)),
            pl.BlockSpec((1, _BLK, D), lambda i: (1, i, 0)),
            pl.BlockSpec((1, _BLK, D), lambda i: (0, i, 0)),
            pl.BlockSpec((_BLK, 2), lambda i: (i, 0)),
            pl.BlockSpec((1, D), lambda i: (0, 0)),
        ],
        out_specs=pl.BlockSpec((_BLK, D), lambda i: (i, 0)),
        out_shape=jax.ShapeDtypeStruct((N, D), jnp.float32),
    )(sp, sp, hp, degp2, b2)


def kernel(x, edge_index, W, b):
    src = edge_index[0].reshape(NW, NCHUNK, CH)
    dst = edge_index[1].reshape(NW, NCHUNK, CH)
    ones_ch = jnp.ones((CH,), jnp.float32)
    zeros_n = jnp.zeros((N,), jnp.float32)
    zeros_rpt = jnp.zeros((RPT, D), jnp.float32)

    h = _tc_matmul(x, W)
    degp = _sc_degree(dst, ones_ch, zeros_n)
    degp2 = degp.reshape(NC, N).T
    hp = _tc_scale(h, degp2)
    sp = _sc_scatter(hp, src, dst, zeros_rpt)
    return _tc_finalize(sp, hp, degp2, b.reshape(1, D))


# speedup vs baseline: 1.1289x; 1.1289x over previous
import functools

import jax
import jax.numpy as jnp
from jax import lax
from jax.experimental import pallas as pl
from jax.experimental.pallas import tpu as pltpu
from jax.experimental.pallas import tpu_sc as plsc

N = 10000
E = 320000
D = 128
NC = 2
NS = 16
NW = NC * NS
EPT = E // NW
CH = 125
NCHUNK = EPT // CH
CH2 = 80
EPT2 = 10080
NCHUNK2 = EPT2 // CH2
NRING = 3
NP = 10240
RPT = NP // NS

_MESH = plsc.VectorSubcoreMesh(core_axis_name="c", subcore_axis_name="s")


def _sc_degree(dst_r, ones_ch, zeros_n):
    @functools.partial(
        pl.kernel,
        mesh=_MESH,
        out_type=jax.ShapeDtypeStruct((NC, 1, N), jnp.float32),
        scratch_types=[
            pltpu.VMEM((NCHUNK, CH), jnp.int32),
            pltpu.VMEM((CH,), jnp.float32),
            pltpu.VMEM_SHARED((N,), jnp.float32),
        ],
    )
    def k(dst_hbm, ones_hbm, zeros_hbm, out_hbm, idx_v, ones_v, acc_s):
        cid = lax.axis_index("c")
        sid = lax.axis_index("s")
        wid = sid * NC + cid

        @pl.when(sid == 0)
        def _():
            pltpu.sync_copy(zeros_hbm, acc_s)

        pltpu.sync_copy(ones_hbm, ones_v)
        pltpu.sync_copy(dst_hbm.at[wid], idx_v)
        plsc.subcore_barrier()

        @pl.loop(0, NCHUNK)
        def _(j):
            pltpu.sync_copy(ones_v, acc_s.at[idx_v.at[j]], add=True)

        plsc.subcore_barrier()

        @pl.when(sid == 0)
        def _():
            pltpu.sync_copy(acc_s, out_hbm.at[cid, 0])

    return k(dst_r, ones_ch, zeros_n)


def _sc_scatter(hp, pe, zeros_rpt):
    @functools.partial(
        pl.kernel,
        mesh=_MESH,
        out_type=jax.ShapeDtypeStruct((NC, NP, D), jnp.float32),
        scratch_types=[
            pltpu.VMEM((EPT2,), jnp.int32),
            pltpu.VMEM((CH2,), jnp.int32),
            pltpu.VMEM((CH2,), jnp.int32),
            pltpu.VMEM((CH2,), jnp.int32),
            pltpu.VMEM((CH2,), jnp.int32),
            pltpu.VMEM((CH2,), jnp.int32),
            pltpu.VMEM((CH2,), jnp.int32),
            pltpu.VMEM((CH2, D), jnp.float32),
            pltpu.VMEM((CH2, D), jnp.float32),
            pltpu.VMEM((CH2, D), jnp.float32),
            pltpu.VMEM_SHARED((NP, D), jnp.float32),
            pltpu.SemaphoreType.DMA,
            pltpu.SemaphoreType.DMA,
        ],
    )
    def k(hp_hbm, pe_hbm, z_hbm, out_hbm,
          pidx, s0, s1, s2, d0, d1, d2, r0, r1, r2, acc, gsem, ssem):
        cid = lax.axis_index("c")
        sid = lax.axis_index("s")
        wid = sid * NC + cid
        base = sid * RPT
        sbufs, dbufs, rbufs = (s0, s1, s2), (d0, d1, d2), (r0, r1, r2)

        pltpu.sync_copy(pe_hbm.at[wid], pidx)
        pltpu.sync_copy(z_hbm, acc.at[pl.ds(base, RPT), :])

        sh14 = jnp.full((16,), 14, jnp.int32)
        m14 = jnp.full((16,), 16383, jnp.int32)

        def unpack(c, t):
            for kk in range(CH2 // 16):
                p = pidx[pl.ds(c * CH2 + kk * 16, 16)]
                sbufs[t][pl.ds(kk * 16, 16)] = lax.shift_right_logical(p, sh14)
                dbufs[t][pl.ds(kk * 16, 16)] = lax.bitwise_and(p, m14)

        def g_start(t):
            pltpu.async_copy(hp_hbm.at[sbufs[t]], rbufs[t], gsem)

        def g_wait(t):
            pltpu.make_async_copy(hp_hbm.at[sbufs[t]], rbufs[t], gsem).wait()

        def s_start(t):
            pltpu.async_copy(rbufs[t], acc.at[dbufs[t]], ssem, add=True)

        def s_wait(t):
            pltpu.make_async_copy(rbufs[t], acc.at[dbufs[t]], ssem).wait()

        plsc.subcore_barrier()

        for t in range(NRING):
            unpack(t, t)
            g_start(t)

        @pl.loop(0, NCHUNK2, step=NRING)
        def _(j):
            for t in range(NRING):
                g_wait(t)
                s_start(t)

            for t in range(NRING):
                @pl.when(j + NRING + t < NCHUNK2)
                def _():
                    s_wait(t)
                    unpack(j + NRING + t, t)
                    g_start(t)

        for t in range(NRING):
            s_wait(t)
        plsc.subcore_barrier()
        pltpu.sync_copy(acc.at[pl.ds(base, RPT), :],
                        out_hbm.at[cid, pl.ds(base, RPT), :])

    return k(hp, pe, zeros_rpt)


_BLK = 2000


def _tc_hprime(x, W, degp2):
    def body(x_ref, w_ref, d_ref, o_ref):
        h = jnp.dot(x_ref[...], w_ref[...], preferred_element_type=jnp.float32)
        deg = 1.0 + d_ref[..., 0] + d_ref[..., 1]
        o_ref[...] = h * lax.rsqrt(deg)[:, None]

    return pl.pallas_call(
        body,
        grid=(N // _BLK,),
        in_specs=[
            pl.BlockSpec((_BLK, D), lambda i: (i, 0)),
            pl.BlockSpec((D, D), lambda i: (0, 0)),
            pl.BlockSpec((_BLK, 2), lambda i: (i, 0)),
        ],
        out_specs=pl.BlockSpec((_BLK, D), lambda i: (i, 0)),
        out_shape=jax.ShapeDtypeStruct((N, D), jnp.float32),
    )(x, W, degp2)


def _tc_finalize(sp, hp, degp2, b2):
    def body(s0_ref, s1_ref, hp_ref, d_ref, b_ref, o_ref):
        deg = 1.0 + d_ref[..., 0] + d_ref[..., 1]
        dinv = lax.rsqrt(deg)[:, None]
        acc = (s0_ref[0] + s1_ref[0] + hp_ref[...]) * dinv + b_ref[...]
        o_ref[...] = jnp.tanh(acc)

    return pl.pallas_call(
        body,
        grid=(N // _BLK,),
        in_specs=[
            pl.BlockSpec((1, _BLK, D), lambda i: (0, i, 0)),
            pl.BlockSpec((1, _BLK, D), lambda i: (1, i, 0)),
            pl.BlockSpec((_BLK, D), lambda i: (i, 0)),
            pl.BlockSpec((_BLK, 2), lambda i: (i, 0)),
            pl.BlockSpec((1, D), lambda i: (0, 0)),
        ],
        out_specs=pl.BlockSpec((_BLK, D), lambda i: (i, 0)),
        out_shape=jax.ShapeDtypeStruct((N, D), jnp.float32),
    )(sp, sp, hp, degp2, b2)


def kernel(x, edge_index, W, b):
    src = edge_index[0]
    dst = edge_index[1]
    dst_r = dst.reshape(NW, NCHUNK, CH)
    ones_ch = jnp.ones((CH,), jnp.float32)
    zeros_n = jnp.zeros((N,), jnp.float32)
    zeros_rpt = jnp.zeros((RPT, D), jnp.float32)

    npad = EPT2 - EPT
    pad_src = (jnp.arange(NW * npad, dtype=jnp.int32) % N).reshape(NW, npad)
    pad_pe = pad_src * 16384 + (NP - 1)
    pe = jnp.concatenate(
        [(src * 16384 + dst).reshape(NW, EPT), pad_pe], axis=1)

    degp = _sc_degree(dst_r, ones_ch, zeros_n)
    degp2 = degp.reshape(NC, N).T
    hp = _tc_hprime(x, W, degp2)
    sp = _sc_scatter(hp, pe, zeros_rpt)
    return _tc_finalize(sp, hp, degp2, b.reshape(1, D))


# speedup vs baseline: 1.1370x; 1.0072x over previous
import functools

import jax
import jax.numpy as jnp
from jax import lax
from jax.experimental import pallas as pl
from jax.experimental.pallas import tpu as pltpu
from jax.experimental.pallas import tpu_sc as plsc

N = 10000
E = 320000
D = 128
NC = 2
NS = 16
NW = NC * NS
EPT = E // NW
CH = 125
NCHUNK = EPT // CH
CH2 = 96
EPT2 = 10080
NCHUNK2 = EPT2 // CH2
NRING = 3
NP = 10240
RPT = NP // NS

_MESH = plsc.VectorSubcoreMesh(core_axis_name="c", subcore_axis_name="s")


def _sc_degree(dst_r, ones_ch, zeros_n):
    @functools.partial(
        pl.kernel,
        mesh=_MESH,
        out_type=jax.ShapeDtypeStruct((NC, 1, N), jnp.float32),
        scratch_types=[
            pltpu.VMEM((NCHUNK, CH), jnp.int32),
            pltpu.VMEM((CH,), jnp.float32),
            pltpu.VMEM_SHARED((N,), jnp.float32),
        ],
    )
    def k(dst_hbm, ones_hbm, zeros_hbm, out_hbm, idx_v, ones_v, acc_s):
        cid = lax.axis_index("c")
        sid = lax.axis_index("s")
        wid = sid * NC + cid

        @pl.when(sid == 0)
        def _():
            pltpu.sync_copy(zeros_hbm, acc_s)

        pltpu.sync_copy(ones_hbm, ones_v)
        pltpu.sync_copy(dst_hbm.at[wid], idx_v)
        plsc.subcore_barrier()

        @pl.loop(0, NCHUNK)
        def _(j):
            pltpu.sync_copy(ones_v, acc_s.at[idx_v.at[j]], add=True)

        plsc.subcore_barrier()

        @pl.when(sid == 0)
        def _():
            pltpu.sync_copy(acc_s, out_hbm.at[cid, 0])

    return k(dst_r, ones_ch, zeros_n)


def _sc_scatter(hp, pe, zeros_rpt):
    @functools.partial(
        pl.kernel,
        mesh=_MESH,
        out_type=jax.ShapeDtypeStruct((NC, NP, D), jnp.float32),
        scratch_types=[
            pltpu.VMEM((EPT2,), jnp.int32),
            pltpu.VMEM((CH2,), jnp.int32),
            pltpu.VMEM((CH2,), jnp.int32),
            pltpu.VMEM((CH2,), jnp.int32),
            pltpu.VMEM((CH2,), jnp.int32),
            pltpu.VMEM((CH2,), jnp.int32),
            pltpu.VMEM((CH2,), jnp.int32),
            pltpu.VMEM((CH2, D), jnp.float32),
            pltpu.VMEM((CH2, D), jnp.float32),
            pltpu.VMEM((CH2, D), jnp.float32),
            pltpu.VMEM_SHARED((NP, D), jnp.float32),
            pltpu.SemaphoreType.DMA,
            pltpu.SemaphoreType.DMA,
        ],
    )
    def k(hp_hbm, pe_hbm, z_hbm, out_hbm,
          pidx, s0, s1, s2, d0, d1, d2, r0, r1, r2, acc, gsem, ssem):
        cid = lax.axis_index("c")
        sid = lax.axis_index("s")
        wid = sid * NC + cid
        base = sid * RPT
        sbufs, dbufs, rbufs = (s0, s1, s2), (d0, d1, d2), (r0, r1, r2)

        pltpu.sync_copy(pe_hbm.at[wid], pidx)
        pltpu.sync_copy(z_hbm, acc.at[pl.ds(base, RPT), :])

        sh14 = jnp.full((16,), 14, jnp.int32)
        m14 = jnp.full((16,), 16383, jnp.int32)

        def unpack(c, t):
            for kk in range(CH2 // 16):
                p = pidx[pl.ds(c * CH2 + kk * 16, 16)]
                sbufs[t][pl.ds(kk * 16, 16)] = lax.shift_right_logical(p, sh14)
                dbufs[t][pl.ds(kk * 16, 16)] = lax.bitwise_and(p, m14)

        def g_start(t):
            pltpu.async_copy(hp_hbm.at[sbufs[t]], rbufs[t], gsem)

        def g_wait(t):
            pltpu.make_async_copy(hp_hbm.at[sbufs[t]], rbufs[t], gsem).wait()

        def s_start(t):
            pltpu.async_copy(rbufs[t], acc.at[dbufs[t]], ssem, add=True)

        def s_wait(t):
            pltpu.make_async_copy(rbufs[t], acc.at[dbufs[t]], ssem).wait()

        plsc.subcore_barrier()

        for t in range(NRING):
            unpack(t, t)
            g_start(t)

        @pl.loop(0, NCHUNK2, step=NRING)
        def _(j):
            for t in range(NRING):
                g_wait(t)
                s_start(t)

            for t in range(NRING):
                @pl.when(j + NRING + t < NCHUNK2)
                def _():
                    s_wait(t)
                    unpack(j + NRING + t, t)
                    g_start(t)

        for t in range(NRING):
            s_wait(t)
        plsc.subcore_barrier()
        pltpu.sync_copy(acc.at[pl.ds(base, RPT), :],
                        out_hbm.at[cid, pl.ds(base, RPT), :])

    return k(hp, pe, zeros_rpt)


_BLK = 2000


def _tc_hprime(x, W, degp2):
    def body(x_ref, w_ref, d_ref, o_ref):
        h = jnp.dot(x_ref[...], w_ref[...], preferred_element_type=jnp.float32)
        deg = 1.0 + d_ref[..., 0] + d_ref[..., 1]
        o_ref[...] = h * lax.rsqrt(deg)[:, None]

    return pl.pallas_call(
        body,
        grid=(N // _BLK,),
        in_specs=[
            pl.BlockSpec((_BLK, D), lambda i: (i, 0)),
            pl.BlockSpec((D, D), lambda i: (0, 0)),
            pl.BlockSpec((_BLK, 2), lambda i: (i, 0)),
        ],
        out_specs=pl.BlockSpec((_BLK, D), lambda i: (i, 0)),
        out_shape=jax.ShapeDtypeStruct((N, D), jnp.float32),
    )(x, W, degp2)


def _tc_finalize(sp, hp, degp2, b2):
    def body(s0_ref, s1_ref, hp_ref, d_ref, b_ref, o_ref):
        deg = 1.0 + d_ref[..., 0] + d_ref[..., 1]
        dinv = lax.rsqrt(deg)[:, None]
        acc = (s0_ref[0] + s1_ref[0] + hp_ref[...]) * dinv + b_ref[...]
        o_ref[...] = jnp.tanh(acc)

    return pl.pallas_call(
        body,
        grid=(N // _BLK,),
        in_specs=[
            pl.BlockSpec((1, _BLK, D), lambda i: (0, i, 0)),
            pl.BlockSpec((1, _BLK, D), lambda i: (1, i, 0)),
            pl.BlockSpec((_BLK, D), lambda i: (i, 0)),
            pl.BlockSpec((_BLK, 2), lambda i: (i, 0)),
            pl.BlockSpec((1, D), lambda i: (0, 0)),
        ],
        out_specs=pl.BlockSpec((_BLK, D), lambda i: (i, 0)),
        out_shape=jax.ShapeDtypeStruct((N, D), jnp.float32),
    )(sp, sp, hp, degp2, b2)


def kernel(x, edge_index, W, b):
    src = edge_index[0]
    dst = edge_index[1]
    dst_r = dst.reshape(NW, NCHUNK, CH)
    ones_ch = jnp.ones((CH,), jnp.float32)
    zeros_n = jnp.zeros((N,), jnp.float32)
    zeros_rpt = jnp.zeros((RPT, D), jnp.float32)

    npad = EPT2 - EPT
    pad_src = (jnp.arange(NW * npad, dtype=jnp.int32) % N).reshape(NW, npad)
    pad_pe = pad_src * 16384 + (NP - 1)
    pe = jnp.concatenate(
        [(src * 16384 + dst).reshape(NW, EPT), pad_pe], axis=1)

    degp = _sc_degree(dst_r, ones_ch, zeros_n)
    degp2 = degp.reshape(NC, N).T
    hp = _tc_hprime(x, W, degp2)
    sp = _sc_scatter(hp, pe, zeros_rpt)
    return _tc_finalize(sp, hp, degp2, b.reshape(1, D))


# speedup vs baseline: 1.1415x; 1.0040x over previous
import functools

import jax
import jax.numpy as jnp
from jax import lax
from jax.experimental import pallas as pl
from jax.experimental.pallas import tpu as pltpu
from jax.experimental.pallas import tpu_sc as plsc

N = 10000
E = 320000
D = 128
NC = 2
NS = 16
NW = NC * NS
EPT = E // NW
CH = 125
NCHUNK = EPT // CH
CH2 = 96
EPT2 = 10080
NCHUNK2 = EPT2 // CH2
NRING = 3
NP = 10240
RPT = NP // NS

def _mesh():
    return plsc.VectorSubcoreMesh(core_axis_name="c", subcore_axis_name="s")


def _sc_degree(dst_r, ones_ch, zeros_n):
    @functools.partial(
        pl.kernel,
        mesh=_mesh(),
        out_type=jax.ShapeDtypeStruct((NC, 1, N), jnp.float32),
        scratch_types=[
            pltpu.VMEM((NCHUNK, CH), jnp.int32),
            pltpu.VMEM((CH,), jnp.float32),
            pltpu.VMEM_SHARED((N,), jnp.float32),
        ],
    )
    def k(dst_hbm, ones_hbm, zeros_hbm, out_hbm, idx_v, ones_v, acc_s):
        cid = lax.axis_index("c")
        sid = lax.axis_index("s")
        wid = sid * NC + cid

        @pl.when(sid == 0)
        def _():
            pltpu.sync_copy(zeros_hbm, acc_s)

        pltpu.sync_copy(ones_hbm, ones_v)
        pltpu.sync_copy(dst_hbm.at[wid], idx_v)
        plsc.subcore_barrier()

        @pl.loop(0, NCHUNK)
        def _(j):
            pltpu.sync_copy(ones_v, acc_s.at[idx_v.at[j]], add=True)

        plsc.subcore_barrier()

        @pl.when(sid == 0)
        def _():
            pltpu.sync_copy(acc_s, out_hbm.at[cid, 0])

    return k(dst_r, ones_ch, zeros_n)


def _sc_scatter(hp, pe, zeros_rpt):
    @functools.partial(
        pl.kernel,
        mesh=_mesh(),
        out_type=jax.ShapeDtypeStruct((NC, NP, D), jnp.float32),
        scratch_types=[
            pltpu.VMEM((EPT2,), jnp.int32),
            pltpu.VMEM((CH2,), jnp.int32),
            pltpu.VMEM((CH2,), jnp.int32),
            pltpu.VMEM((CH2,), jnp.int32),
            pltpu.VMEM((CH2,), jnp.int32),
            pltpu.VMEM((CH2,), jnp.int32),
            pltpu.VMEM((CH2,), jnp.int32),
            pltpu.VMEM((CH2, D), jnp.float32),
            pltpu.VMEM((CH2, D), jnp.float32),
            pltpu.VMEM((CH2, D), jnp.float32),
            pltpu.VMEM_SHARED((NP, D), jnp.float32),
            pltpu.SemaphoreType.DMA,
            pltpu.SemaphoreType.DMA,
        ],
    )
    def k(hp_hbm, pe_hbm, z_hbm, out_hbm,
          pidx, s0, s1, s2, d0, d1, d2, r0, r1, r2, acc, gsem, ssem):
        cid = lax.axis_index("c")
        sid = lax.axis_index("s")
        wid = sid * NC + cid
        base = sid * RPT
        sbufs, dbufs, rbufs = (s0, s1, s2), (d0, d1, d2), (r0, r1, r2)

        pltpu.sync_copy(pe_hbm.at[wid], pidx)
        pltpu.sync_copy(z_hbm, acc.at[pl.ds(base, RPT), :])

        sh14 = jnp.full((16,), 14, jnp.int32)
        m14 = jnp.full((16,), 16383, jnp.int32)

        def unpack(c, t):
            for kk in range(CH2 // 16):
                p = pidx[pl.ds(c * CH2 + kk * 16, 16)]
                sbufs[t][pl.ds(kk * 16, 16)] = lax.shift_right_logical(p, sh14)
                dbufs[t][pl.ds(kk * 16, 16)] = lax.bitwise_and(p, m14)

        def g_start(t):
            pltpu.async_copy(hp_hbm.at[sbufs[t]], rbufs[t], gsem)

        def g_wait(t):
            pltpu.make_async_copy(hp_hbm.at[sbufs[t]], rbufs[t], gsem).wait()

        def s_start(t):
            pltpu.async_copy(rbufs[t], acc.at[dbufs[t]], ssem, add=True)

        def s_wait(t):
            pltpu.make_async_copy(rbufs[t], acc.at[dbufs[t]], ssem).wait()

        plsc.subcore_barrier()

        for t in range(NRING):
            unpack(t, t)
            g_start(t)

        @pl.loop(0, NCHUNK2, step=NRING)
        def _(j):
            for t in range(NRING):
                g_wait(t)
                s_start(t)

            for t in range(NRING):
                @pl.when(j + NRING + t < NCHUNK2)
                def _():
                    s_wait(t)
                    unpack(j + NRING + t, t)
                    g_start(t)

        for t in range(NRING):
            s_wait(t)
        plsc.subcore_barrier()
        pltpu.sync_copy(acc.at[pl.ds(base, RPT), :],
                        out_hbm.at[cid, pl.ds(base, RPT), :])

    return k(hp, pe, zeros_rpt)


_BLK = 2000


def _tc_hprime(x, W, degp2):
    def body(x_ref, w_ref, d_ref, o_ref):
        h = jnp.dot(x_ref[...], w_ref[...], preferred_element_type=jnp.float32)
        deg = 1.0 + d_ref[..., 0] + d_ref[..., 1]
        o_ref[...] = h * lax.rsqrt(deg)[:, None]

    return pl.pallas_call(
        body,
        grid=(N // _BLK,),
        in_specs=[
            pl.BlockSpec((_BLK, D), lambda i: (i, 0)),
            pl.BlockSpec((D, D), lambda i: (0, 0)),
            pl.BlockSpec((_BLK, 2), lambda i: (i, 0)),
        ],
        out_specs=pl.BlockSpec((_BLK, D), lambda i: (i, 0)),
        out_shape=jax.ShapeDtypeStruct((N, D), jnp.float32),
    )(x, W, degp2)


def _tc_finalize(sp, hp, degp2, b2):
    def body(s0_ref, s1_ref, hp_ref, d_ref, b_ref, o_ref):
        deg = 1.0 + d_ref[..., 0] + d_ref[..., 1]
        dinv = lax.rsqrt(deg)[:, None]
        acc = (s0_ref[0] + s1_ref[0] + hp_ref[...]) * dinv + b_ref[...]
        o_ref[...] = jnp.tanh(acc)

    return pl.pallas_call(
        body,
        grid=(N // _BLK,),
        in_specs=[
            pl.BlockSpec((1, _BLK, D), lambda i: (0, i, 0)),
            pl.BlockSpec((1, _BLK, D), lambda i: (1, i, 0)),
            pl.BlockSpec((_BLK, D), lambda i: (i, 0)),
            pl.BlockSpec((_BLK, 2), lambda i: (i, 0)),
            pl.BlockSpec((1, D), lambda i: (0, 0)),
        ],
        out_specs=pl.BlockSpec((_BLK, D), lambda i: (i, 0)),
        out_shape=jax.ShapeDtypeStruct((N, D), jnp.float32),
    )(sp, sp, hp, degp2, b2)


def kernel(x, edge_index, W, b):
    src = edge_index[0]
    dst = edge_index[1]
    dst_r = dst.reshape(NW, NCHUNK, CH)
    ones_ch = jnp.ones((CH,), jnp.float32)
    zeros_n = jnp.zeros((N,), jnp.float32)
    zeros_rpt = jnp.zeros((RPT, D), jnp.float32)

    npad = EPT2 - EPT
    pad_src = (jnp.arange(NW * npad, dtype=jnp.int32) % N).reshape(NW, npad)
    pad_pe = pad_src * 16384 + (NP - 1)
    pe = jnp.concatenate(
        [(src * 16384 + dst).reshape(NW, EPT), pad_pe], axis=1)

    degp = _sc_degree(dst_r, ones_ch, zeros_n)
    degp2 = degp.reshape(NC, N).T
    hp = _tc_hprime(x, W, degp2)
    sp = _sc_scatter(hp, pe, zeros_rpt)
    return _tc_finalize(sp, hp, degp2, b.reshape(1, D))
